# baseline (speedup 1.0000x reference)
import functools
import jax
import jax.numpy as jnp
from jax import lax
from jax.experimental import pallas as pl
from jax.experimental.pallas import tpu as pltpu
from jax.experimental.pallas import tpu_sc as plsc

LANES = 128
GATHERS_PER_CHUNK = 2
CHUNK = LANES * GATHERS_PER_CHUNK
NBUF = 3
NIB = 6
PF = 3
UNROLL = 6


@functools.cache
def _build(n_rows: int, vocab: int, d: int):
  info = plsc.get_sparse_core_info()
  nw = info.num_cores * info.num_subcores
  per_w = n_rows // nw
  n_chunks = per_w // CHUNK
  n_groups = n_chunks // UNROLL
  assert n_rows == nw * n_chunks * CHUNK
  assert n_groups >= 2 and n_chunks >= 2 * UNROLL
  mesh = plsc.VectorSubcoreMesh(core_axis_name="c", subcore_axis_name="s")

  @functools.partial(
      pl.kernel,
      mesh=mesh,
      out_type=jax.ShapeDtypeStruct((n_rows, d), jnp.float32),
      scratch_types=[
          pltpu.VMEM((NIB * GATHERS_PER_CHUNK, LANES), jnp.int32),
          pltpu.VMEM((NBUF, CHUNK, d), jnp.float32),
          pltpu.VMEM_SHARED((vocab, d), jnp.float32),
          pltpu.SemaphoreType.DMA((NIB,)),
          pltpu.SemaphoreType.DMA((NBUF,)),
          pltpu.SemaphoreType.DMA((NBUF,)),
      ],
  )
  def k(table_hbm, idx_hbm, out_hbm, idx_v, rows_v, table_sp, isem, gsem,
        wsem):
    wid = lax.axis_index("s") * info.num_cores + lax.axis_index("c")
    base = wid * per_w

    @pl.when(lax.axis_index("s") == 0)
    def _stage():
      pltpu.sync_copy(table_hbm, table_sp)

    def idx_start(c, ib):
      for j in range(GATHERS_PER_CHUNK):
        pltpu.async_copy(
            idx_hbm.at[pl.ds(base + c * CHUNK + j * LANES, LANES)],
            idx_v.at[ib * GATHERS_PER_CHUNK + j], isem.at[ib])

    def idx_wait(c, ib):
      for j in range(GATHERS_PER_CHUNK):
        pltpu.make_async_copy(
            idx_hbm.at[pl.ds(base + c * CHUNK + j * LANES, LANES)],
            idx_v.at[ib * GATHERS_PER_CHUNK + j], isem.at[ib]).wait()

    def gather_start(b, ib):
      for j in range(GATHERS_PER_CHUNK):
        pltpu.async_copy(
            table_sp.at[idx_v.at[ib * GATHERS_PER_CHUNK + j]],
            rows_v.at[b, pl.ds(j * LANES, LANES)], gsem.at[b])

    def gather_wait(b, ib):
      for j in range(GATHERS_PER_CHUNK):
        pltpu.make_async_copy(
            table_sp.at[idx_v.at[ib * GATHERS_PER_CHUNK + j]],
            rows_v.at[b, pl.ds(j * LANES, LANES)], gsem.at[b]).wait()

    def write_start(c, b):
      pltpu.async_copy(rows_v.at[b],
                       out_hbm.at[pl.ds(base + c * CHUNK, CHUNK)],
                       wsem.at[b])

    def write_wait(c, b):
      pltpu.make_async_copy(rows_v.at[b],
                            out_hbm.at[pl.ds(base + c * CHUNK, CHUNK)],
                            wsem.at[b]).wait()

    def step(c, u, first_round):
      b = u % NBUF
      ib = u % NIB
      if not first_round:
        write_wait(c - NBUF, b)
      elif u >= NBUF:
        write_wait(c - NBUF, b)
      idx_wait(c, ib)
      gather_start(b, ib)
      if isinstance(c, int):
        if c + PF < n_chunks:
          idx_start(c + PF, (u + PF) % NIB)
      else:
        @pl.when(c + PF < n_chunks)
        def _():
          idx_start(c + PF, (u + PF) % NIB)
      if not first_round or u >= 1:
        gather_wait((u - 1) % NBUF, (u - 1) % NIB)
        write_start(c - 1, (u - 1) % NBUF)

    for c in range(PF):
      idx_start(c, c % NIB)
    plsc.subcore_barrier()
    for u in range(UNROLL):
      step(u, u, True)

    def body(g, carry):
      c0 = g * UNROLL
      for u in range(UNROLL):
        step(c0 + u, u, False)
      return carry

    lax.fori_loop(1, n_groups, body, 0)

    for c in range(n_groups * UNROLL, n_chunks):
      step(c, c % UNROLL, False)

    last = n_chunks - 1
    gather_wait(last % NBUF, last % NIB)
    write_start(last, last % NBUF)
    for c in range(n_chunks - NBUF, n_chunks):
      write_wait(c, c % NBUF)

  return k


def kernel(x, table):
  b, h = x.shape
  v, d = table.shape
  n = b * h
  x_flat = x.reshape(n).astype(jnp.int32)
  out = _build(n, v, d)(table, x_flat)
  return out.reshape(b, h, d)
